# baseline (speedup 1.0000x reference)
import jax
import jax.numpy as jnp
from jax import lax
from jax.experimental import pallas as pl
from jax.experimental.pallas import tpu as pltpu
from jax.experimental.pallas import tpu_sc as plsc

B = 16384
F = 26
D2 = 16
N = B * F
V = 1_000_000
NC = 2
NS = 16
NW = NC * NS
QF = V // 128
VT = QF * 128
TAIL = V - VT
BPW = B // NW
QW = BPW // 128

C2, C4, C6, C8 = -0.5, 1.0 / 24, -1.0 / 720, 1.0 / 40320
S3, S5, S7, S9 = -1.0 / 6, 1.0 / 120, -1.0 / 5040, 1.0 / 362880

CC0, CC1, CC2, CC3, CC4 = (
    0.99995902, -0.49979061, 0.041494742, -0.0013390585, 1.8781330e-05
)
SS0, SS1, SS2, SS3, SS4 = (
    0.99999615, -0.16664703, 0.0083172454, -1.9376590e-04, 2.1981252e-06
)


def _detile_body(
    pt_hbm, tail_hbm, tab_hbm, tin, tout0, tout1, tailv, sg0, sg1, so0, so1
):
    tout = (tout0, tout1)
    wid = lax.axis_index("s") * NC + lax.axis_index("c")
    start = wid * 244 + jnp.minimum(wid, 4)
    nblk = 244 + (wid < 4).astype(jnp.int32)

    lane = lax.iota(jnp.int32, 16)
    idxjs = [lane * 16 + j for j in range(16)]
    sgs = (sg0, sg1)
    sos = (so0, so1)

    def rd_refs(t):
        q2 = start + t
        return pt_hbm.at[pl.ds(0, 16), pl.ds(q2 * 128, 128)]

    def wr_refs(t):
        q2 = start + t
        return tab_hbm.at[pl.ds(q2 * 2048, 2048)]

    pltpu.async_copy(rd_refs(0), tin.at[0], sg0)
    pltpu.async_copy(rd_refs(1), tin.at[1], sg1)

    @pl.loop(0, 123)
    def _(g):
        for p in range(2):
            t = 2 * g + p

            @pl.when(t < nblk)
            def _(t=t, p=p):
                pltpu.make_async_copy(rd_refs(t), tin.at[p], sgs[p]).wait()
                for a in range(8):
                    dst = tout[p].at[pl.ds(256 * a, 256)]
                    for j in range(16):
                        v = tin[p, j, pl.ds(16 * a, 16)]
                        plsc.store_scatter(dst, [idxjs[j]], v)

                @pl.when(t >= 2)
                def _():
                    pltpu.make_async_copy(
                        tout[p], wr_refs(t - 2), sos[p]
                    ).wait()

                pltpu.async_copy(tout[p], wr_refs(t), sos[p])

                @pl.when(t + 2 < nblk)
                def _():
                    pltpu.async_copy(rd_refs(t + 2), tin.at[p], sgs[p])

    for p in range(2):
        pltpu.make_async_copy(
            tout[p], tab_hbm.at[pl.ds(0, 2048)], sos[p]
        ).wait()

    @pl.when(wid == NW - 1)
    def _():
        pltpu.sync_copy(tail_hbm, tailv)
        pltpu.sync_copy(tailv, tab_hbm.at[pl.ds(VT * D2, TAIL * D2)])


_sc_detile = pl.kernel(
    _detile_body,
    out_type=jax.ShapeDtypeStruct((V * D2,), jnp.float32),
    mesh=plsc.VectorSubcoreMesh(core_axis_name="c", subcore_axis_name="s"),
    compiler_params=pltpu.CompilerParams(
        needs_layout_passes=False, use_tc_tiling_on_sc=True
    ),
    scratch_types=[
        pltpu.VMEM((2, 16, 128), jnp.float32),
        pltpu.VMEM((2048,), jnp.float32),
        pltpu.VMEM((2048,), jnp.float32),
        pltpu.VMEM((TAIL * D2,), jnp.float32),
        pltpu.SemaphoreType.DMA,
        pltpu.SemaphoreType.DMA,
        pltpu.SemaphoreType.DMA,
        pltpu.SemaphoreType.DMA,
    ],
)


def _embed_body(
    idx_hbm, tab_hbm, out_hbm, idx_v, rows_v, out_v0, out_v1,
    sg0, sg1, sg2, sg3, so0, so1
):
    wid = lax.axis_index("s") * NC + lax.axis_index("c")

    lane = lax.iota(jnp.int32, 16)
    tblc = (lane // 4) * 4096 + (lane % 4) * 256

    sgs = (sg0, sg1, sg2, sg3)
    sos = (so0, so1)
    outs = (out_v0, out_v1)

    def start_chunk(f):
        p = f % 4
        pltpu.sync_copy(idx_hbm.at[f, pl.ds(QW * wid, QW)], idx_v.at[p])
        return [
            pltpu.async_copy(
                tab_hbm.at[idx_v.at[p, c]],
                rows_v.at[p, pl.ds(c * 128, 128)],
                sgs[p],
            )
            for c in range(QW)
        ]

    gathers = {f: start_chunk(f) for f in range(3)}
    out_copies = {}
    for f in range(F):
        p = f % 4
        q = f % 2
        if f + 3 < F:
            gathers[f + 3] = start_chunk(f + 3)
        for c in gathers.pop(f):
            c.wait()
        if f >= 2:
            for c in out_copies.pop(f - 2):
                c.wait()

        @plsc.parallel_loop(0, BPW, step=1, unroll=4)
        def _(i, p=p, q=q):
            th = rows_v[p, i, :]
            h = th * 0.5
            u = h * h
            c = CC0 + u * (CC1 + u * (CC2 + u * (CC3 + u * CC4)))
            s = h * (SS0 + u * (SS1 + u * (SS2 + u * (SS3 + u * SS4))))
            cb = 2.0 * c * c - 1.0
            sb = 2.0 * s * c
            base = 8 * i - 7 * (i & 127)
            idxc = tblc + base
            plsc.store_scatter(outs[q], [idxc], cb)
            plsc.store_scatter(outs[q], [idxc + 128], sb)

        obase = f * (B * 32) + wid * 4096
        out_copies[f] = [
            pltpu.async_copy(
                outs[q].at[pl.ds(kt * 4096, 4096)],
                out_hbm.at[pl.ds(obase + kt * (128 * 1024), 4096)],
                sos[q],
            )
            for kt in range(4)
        ]
    for f in sorted(out_copies):
        for c in out_copies[f]:
            c.wait()


_sc_embed = pl.kernel(
    _embed_body,
    out_type=jax.ShapeDtypeStruct((N * 32,), jnp.float32),
    mesh=plsc.VectorSubcoreMesh(core_axis_name="c", subcore_axis_name="s"),
    compiler_params=pltpu.CompilerParams(
        needs_layout_passes=False, use_tc_tiling_on_sc=False
    ),
    scratch_types=[
        pltpu.VMEM((4, QW, 128), jnp.int32),
        pltpu.VMEM((4, BPW, D2), jnp.float32),
        pltpu.VMEM((32 * 512,), jnp.float32),
        pltpu.VMEM((32 * 512,), jnp.float32),
        pltpu.SemaphoreType.DMA,
        pltpu.SemaphoreType.DMA,
        pltpu.SemaphoreType.DMA,
        pltpu.SemaphoreType.DMA,
        pltpu.SemaphoreType.DMA,
        pltpu.SemaphoreType.DMA,
    ],
)


def kernel(input, phases):
    phases_t = phases.T
    tail = phases[VT:, :].reshape(TAIL * D2)
    table = _sc_detile(phases_t, tail)
    idx3 = input.T.reshape(F, 128, 128).astype(jnp.int32)
    flat = _sc_embed(idx3, table.reshape(V, D2))
    out = flat.reshape(F, 4, 128, 8, 128).transpose(2, 4, 0, 1, 3)
    return out.reshape(B, F, 32)


# speedup vs baseline: 1.1079x; 1.1079x over previous
import jax
import jax.numpy as jnp
from jax import lax
from jax.experimental import pallas as pl
from jax.experimental.pallas import tpu as pltpu
from jax.experimental.pallas import tpu_sc as plsc

B = 16384
F = 26
D2 = 16
N = B * F
V = 1_000_000
NC = 2
NS = 16
NW = NC * NS
QF = V // 128
VT = QF * 128
TAIL = V - VT
BPW = B // NW
QW = BPW // 128

C2, C4, C6, C8 = -0.5, 1.0 / 24, -1.0 / 720, 1.0 / 40320
S3, S5, S7, S9 = -1.0 / 6, 1.0 / 120, -1.0 / 5040, 1.0 / 362880

CC0, CC1, CC2, CC3, CC4 = (
    0.99995902, -0.49979061, 0.041494742, -0.0013390585, 1.8781330e-05
)
SS0, SS1, SS2, SS3, SS4 = (
    0.99999615, -0.16664703, 0.0083172454, -1.9376590e-04, 2.1981252e-06
)


def _detile_body(
    pt_hbm, tail_hbm, tab_hbm, tin, tout0, tout1, tailv, sg0, sg1, so0, so1
):
    tout = (tout0, tout1)
    wid = lax.axis_index("s") * NC + lax.axis_index("c")
    start = wid * 61 + jnp.minimum(wid, 1)
    nblk = 61 + (wid < 1).astype(jnp.int32)

    lane = lax.iota(jnp.int32, 16)
    idxjs = [lane * 16 + j for j in range(16)]
    sgs = (sg0, sg1)
    sos = (so0, so1)

    def rd_refs(t):
        q2 = start + t
        return pt_hbm.at[pl.ds(0, 16), pl.ds(q2 * 512, 512)]

    def wr_refs(t):
        q2 = start + t
        return tab_hbm.at[pl.ds(q2 * 8192, 8192)]

    pltpu.async_copy(rd_refs(0), tin.at[0], sg0)
    pltpu.async_copy(rd_refs(1), tin.at[1], sg1)

    @pl.loop(0, 31)
    def _(g):
        for p in range(2):
            t = 2 * g + p

            @pl.when(t < nblk)
            def _(t=t, p=p):
                pltpu.make_async_copy(rd_refs(t), tin.at[p], sgs[p]).wait()

                @pl.loop(0, 4)
                def _(gr, p=p):
                    dst = tout[p].at[pl.ds(2048 * gr, 2048)]
                    for a in range(8):
                        for j in range(16):
                            v = tin[p, j, pl.ds(128 * gr + 16 * a, 16)]
                            plsc.store_scatter(
                                dst.at[pl.ds(256 * a, 256)], [idxjs[j]], v
                            )

                @pl.when(t >= 2)
                def _():
                    pltpu.make_async_copy(
                        tout[p], wr_refs(t - 2), sos[p]
                    ).wait()

                pltpu.async_copy(tout[p], wr_refs(t), sos[p])

                @pl.when(t + 2 < nblk)
                def _():
                    pltpu.async_copy(rd_refs(t + 2), tin.at[p], sgs[p])

    for p in range(2):
        pltpu.make_async_copy(
            tout[p], tab_hbm.at[pl.ds(0, 8192)], sos[p]
        ).wait()

    @pl.when(wid == NW - 1)
    def _():
        pltpu.sync_copy(tail_hbm, tailv)
        pltpu.sync_copy(tailv, tab_hbm.at[pl.ds(VT * D2, TAIL * D2)])


_sc_detile = pl.kernel(
    _detile_body,
    out_type=jax.ShapeDtypeStruct((V * D2,), jnp.float32),
    mesh=plsc.VectorSubcoreMesh(core_axis_name="c", subcore_axis_name="s"),
    compiler_params=pltpu.CompilerParams(
        needs_layout_passes=False, use_tc_tiling_on_sc=True
    ),
    scratch_types=[
        pltpu.VMEM((2, 16, 512), jnp.float32),
        pltpu.VMEM((8192,), jnp.float32),
        pltpu.VMEM((8192,), jnp.float32),
        pltpu.VMEM((TAIL * D2,), jnp.float32),
        pltpu.SemaphoreType.DMA,
        pltpu.SemaphoreType.DMA,
        pltpu.SemaphoreType.DMA,
        pltpu.SemaphoreType.DMA,
    ],
)


def _embed_body(
    idx_hbm, tab_hbm, out_hbm, idx_v, rows_v, out_v0, out_v1,
    sg0, sg1, sg2, sg3, so0, so1
):
    wid = lax.axis_index("s") * NC + lax.axis_index("c")

    lane = lax.iota(jnp.int32, 16)
    tblc = (lane // 4) * 4096 + (lane % 4) * 256

    sgs = (sg0, sg1, sg2, sg3)
    sos = (so0, so1)
    outs = (out_v0, out_v1)

    def start_chunk(f):
        p = f % 4
        pltpu.sync_copy(idx_hbm.at[f, pl.ds(QW * wid, QW)], idx_v.at[p])
        return [
            pltpu.async_copy(
                tab_hbm.at[idx_v.at[p, c]],
                rows_v.at[p, pl.ds(c * 128, 128)],
                sgs[p],
            )
            for c in range(QW)
        ]

    gathers = {f: start_chunk(f) for f in range(3)}
    out_copies = {}
    for f in range(F):
        p = f % 4
        q = f % 2
        if f + 3 < F:
            gathers[f + 3] = start_chunk(f + 3)
        for c in gathers.pop(f):
            c.wait()
        if f >= 2:
            for c in out_copies.pop(f - 2):
                c.wait()

        @plsc.parallel_loop(0, BPW, step=1, unroll=4)
        def _(i, p=p, q=q):
            th = rows_v[p, i, :]
            h = th * 0.5
            u = h * h
            c = CC0 + u * (CC1 + u * (CC2 + u * (CC3 + u * CC4)))
            s = h * (SS0 + u * (SS1 + u * (SS2 + u * (SS3 + u * SS4))))
            cb = 2.0 * c * c - 1.0
            sb = 2.0 * s * c
            base = 8 * i - 7 * (i & 127)
            idxc = tblc + base
            plsc.store_scatter(outs[q], [idxc], cb)
            plsc.store_scatter(outs[q], [idxc + 128], sb)

        obase = f * (B * 32) + wid * 4096
        out_copies[f] = [
            pltpu.async_copy(
                outs[q].at[pl.ds(kt * 4096, 4096)],
                out_hbm.at[pl.ds(obase + kt * (128 * 1024), 4096)],
                sos[q],
            )
            for kt in range(4)
        ]
    for f in sorted(out_copies):
        for c in out_copies[f]:
            c.wait()


_sc_embed = pl.kernel(
    _embed_body,
    out_type=jax.ShapeDtypeStruct((N * 32,), jnp.float32),
    mesh=plsc.VectorSubcoreMesh(core_axis_name="c", subcore_axis_name="s"),
    compiler_params=pltpu.CompilerParams(
        needs_layout_passes=False, use_tc_tiling_on_sc=False
    ),
    scratch_types=[
        pltpu.VMEM((4, QW, 128), jnp.int32),
        pltpu.VMEM((4, BPW, D2), jnp.float32),
        pltpu.VMEM((32 * 512,), jnp.float32),
        pltpu.VMEM((32 * 512,), jnp.float32),
        pltpu.SemaphoreType.DMA,
        pltpu.SemaphoreType.DMA,
        pltpu.SemaphoreType.DMA,
        pltpu.SemaphoreType.DMA,
        pltpu.SemaphoreType.DMA,
        pltpu.SemaphoreType.DMA,
    ],
)


def kernel(input, phases):
    phases_t = phases.T
    tail = phases[VT:, :].reshape(TAIL * D2)
    table = _sc_detile(phases_t, tail)
    idx3 = input.T.reshape(F, 128, 128).astype(jnp.int32)
    flat = _sc_embed(idx3, table.reshape(V, D2))
    out = flat.reshape(F, 4, 128, 8, 128).transpose(2, 4, 0, 1, 3)
    return out.reshape(B, F, 32)


# speedup vs baseline: 1.1101x; 1.0019x over previous
import jax
import jax.numpy as jnp
from jax import lax
from jax.experimental import pallas as pl
from jax.experimental.pallas import tpu as pltpu
from jax.experimental.pallas import tpu_sc as plsc

B = 16384
F = 26
D2 = 16
N = B * F
V = 1_000_000
NC = 2
NS = 16
NW = NC * NS
QB = V // 1024
VT = QB * 1024
TAIL = V - VT
BPW = B // NW
QW = BPW // 128

C2, C4, C6, C8 = -0.5, 1.0 / 24, -1.0 / 720, 1.0 / 40320
S3, S5, S7, S9 = -1.0 / 6, 1.0 / 120, -1.0 / 5040, 1.0 / 362880

CC0, CC1, CC2, CC3, CC4 = (
    0.99995902, -0.49979061, 0.041494742, -0.0013390585, 1.8781330e-05
)
SS0, SS1, SS2, SS3, SS4 = (
    0.99999615, -0.16664703, 0.0083172454, -1.9376590e-04, 2.1981252e-06
)


def _detile_body(
    pt_hbm, tail_hbm, tab_hbm, tin, tout0, tout1, tailv, sg0, sg1, so0, so1
):
    tout = (tout0, tout1)
    wid = lax.axis_index("s") * NC + lax.axis_index("c")
    start = wid * 30 + jnp.minimum(wid, 16)
    nblk = 30 + (wid < 16).astype(jnp.int32)

    lane = lax.iota(jnp.int32, 16)
    idxjs = [lane * 16 + j for j in range(16)]
    sgs = (sg0, sg1)
    sos = (so0, so1)

    def rd_refs(t):
        q2 = start + t
        return pt_hbm.at[pl.ds(0, 16), pl.ds(q2 * 1024, 1024)]

    def wr_refs(t):
        q2 = start + t
        return tab_hbm.at[pl.ds(q2 * 16384, 16384)]

    pltpu.async_copy(rd_refs(0), tin.at[0], sg0)
    pltpu.async_copy(rd_refs(1), tin.at[1], sg1)

    @pl.loop(0, 16)
    def _(g):
        for p in range(2):
            t = 2 * g + p

            @pl.when(t < nblk)
            def _(t=t, p=p):
                pltpu.make_async_copy(rd_refs(t), tin.at[p], sgs[p]).wait()

                @pl.loop(0, 8)
                def _(gr, p=p):
                    dst = tout[p].at[pl.ds(2048 * gr, 2048)]
                    for a in range(8):
                        for j in range(16):
                            v = tin[p, j, pl.ds(128 * gr + 16 * a, 16)]
                            plsc.store_scatter(
                                dst.at[pl.ds(256 * a, 256)], [idxjs[j]], v
                            )

                @pl.when(t >= 2)
                def _():
                    pltpu.make_async_copy(
                        tout[p], wr_refs(t - 2), sos[p]
                    ).wait()

                pltpu.async_copy(tout[p], wr_refs(t), sos[p])

                @pl.when(t + 2 < nblk)
                def _():
                    pltpu.async_copy(rd_refs(t + 2), tin.at[p], sgs[p])

    for p in range(2):
        pltpu.make_async_copy(
            tout[p], tab_hbm.at[pl.ds(0, 16384)], sos[p]
        ).wait()

    @pl.when(wid == NW - 1)
    def _():
        pltpu.sync_copy(tail_hbm, tailv)
        pltpu.sync_copy(tailv, tab_hbm.at[pl.ds(VT * D2, TAIL * D2)])


_sc_detile = pl.kernel(
    _detile_body,
    out_type=jax.ShapeDtypeStruct((V * D2,), jnp.float32),
    mesh=plsc.VectorSubcoreMesh(core_axis_name="c", subcore_axis_name="s"),
    compiler_params=pltpu.CompilerParams(
        needs_layout_passes=False, use_tc_tiling_on_sc=True
    ),
    scratch_types=[
        pltpu.VMEM((2, 16, 1024), jnp.float32),
        pltpu.VMEM((16384,), jnp.float32),
        pltpu.VMEM((16384,), jnp.float32),
        pltpu.VMEM((TAIL * D2,), jnp.float32),
        pltpu.SemaphoreType.DMA,
        pltpu.SemaphoreType.DMA,
        pltpu.SemaphoreType.DMA,
        pltpu.SemaphoreType.DMA,
    ],
)


def _embed_body(
    idx_hbm, tab_hbm, out_hbm, idx_v, rows_v, out_v0, out_v1,
    sg0, sg1, sg2, sg3, so0, so1
):
    wid = lax.axis_index("s") * NC + lax.axis_index("c")

    lane = lax.iota(jnp.int32, 16)
    tblc = (lane // 4) * 4096 + (lane % 4) * 256

    sgs = (sg0, sg1, sg2, sg3)
    sos = (so0, so1)
    outs = (out_v0, out_v1)

    def start_chunk(f):
        p = f % 4
        pltpu.sync_copy(idx_hbm.at[f, pl.ds(QW * wid, QW)], idx_v.at[p])
        return [
            pltpu.async_copy(
                tab_hbm.at[idx_v.at[p, c]],
                rows_v.at[p, pl.ds(c * 128, 128)],
                sgs[p],
            )
            for c in range(QW)
        ]

    gathers = {f: start_chunk(f) for f in range(3)}
    out_copies = {}
    for f in range(F):
        p = f % 4
        q = f % 2
        if f + 3 < F:
            gathers[f + 3] = start_chunk(f + 3)
        for c in gathers.pop(f):
            c.wait()
        if f >= 2:
            for c in out_copies.pop(f - 2):
                c.wait()

        @plsc.parallel_loop(0, BPW, step=1, unroll=4)
        def _(i, p=p, q=q):
            th = rows_v[p, i, :]
            h = th * 0.5
            u = h * h
            c = CC0 + u * (CC1 + u * (CC2 + u * (CC3 + u * CC4)))
            s = h * (SS0 + u * (SS1 + u * (SS2 + u * (SS3 + u * SS4))))
            cb = 2.0 * c * c - 1.0
            sb = 2.0 * s * c
            base = 8 * i - 7 * (i & 127)
            idxc = tblc + base
            plsc.store_scatter(outs[q], [idxc], cb)
            plsc.store_scatter(outs[q], [idxc + 128], sb)

        obase = f * (B * 32) + wid * 4096
        out_copies[f] = [
            pltpu.async_copy(
                outs[q].at[pl.ds(kt * 4096, 4096)],
                out_hbm.at[pl.ds(obase + kt * (128 * 1024), 4096)],
                sos[q],
            )
            for kt in range(4)
        ]
    for f in sorted(out_copies):
        for c in out_copies[f]:
            c.wait()


_sc_embed = pl.kernel(
    _embed_body,
    out_type=jax.ShapeDtypeStruct((N * 32,), jnp.float32),
    mesh=plsc.VectorSubcoreMesh(core_axis_name="c", subcore_axis_name="s"),
    compiler_params=pltpu.CompilerParams(
        needs_layout_passes=False, use_tc_tiling_on_sc=False
    ),
    scratch_types=[
        pltpu.VMEM((4, QW, 128), jnp.int32),
        pltpu.VMEM((4, BPW, D2), jnp.float32),
        pltpu.VMEM((32 * 512,), jnp.float32),
        pltpu.VMEM((32 * 512,), jnp.float32),
        pltpu.SemaphoreType.DMA,
        pltpu.SemaphoreType.DMA,
        pltpu.SemaphoreType.DMA,
        pltpu.SemaphoreType.DMA,
        pltpu.SemaphoreType.DMA,
        pltpu.SemaphoreType.DMA,
    ],
)


def kernel(input, phases):
    phases_t = phases.T
    tail = phases[VT:, :].reshape(TAIL * D2)
    table = _sc_detile(phases_t, tail)
    idx3 = input.T.reshape(F, 128, 128).astype(jnp.int32)
    flat = _sc_embed(idx3, table.reshape(V, D2))
    out = flat.reshape(F, 4, 128, 8, 128).transpose(2, 4, 0, 1, 3)
    return out.reshape(B, F, 32)
